# baseline (speedup 1.0000x reference)
import jax
import jax.numpy as jnp
from jax.experimental import pallas as pl
from jax.experimental.pallas import tpu as pltpu

_BN = 2048


def _body(sid_ref, pts_ref, feat_ref, out_ref):
    bn = out_ref.shape[2]
    lat = feat_ref[0, 0]
    lat_rep = jnp.broadcast_to(lat[:, None], (lat.shape[0], bn))
    out_ref[0] = jnp.concatenate([pts_ref[0], lat_rep], axis=0)


def kernel(points, subject_garment_id, features):
    b, c, n = points.shape
    s, l = features.shape
    grid = (b, n // _BN)
    feats3 = features.reshape(s, 1, l)

    return pl.pallas_call(
        _body,
        grid_spec=pltpu.PrefetchScalarGridSpec(
            num_scalar_prefetch=1,
            grid=grid,
            in_specs=[
                pl.BlockSpec((1, c, _BN), lambda bi, ni, sid: (bi, 0, ni)),
                pl.BlockSpec((1, 1, l), lambda bi, ni, sid: (sid[bi], 0, 0)),
            ],
            out_specs=pl.BlockSpec((1, c + l, _BN), lambda bi, ni, sid: (bi, 0, ni)),
        ),
        out_shape=jax.ShapeDtypeStruct((b, c + l, n), jnp.float32),
    )(subject_garment_id, points, feats3)


# speedup vs baseline: 1.0962x; 1.0962x over previous
import jax
import jax.numpy as jnp
from jax.experimental import pallas as pl
from jax.experimental.pallas import tpu as pltpu

_BN = 8192


def _body(sid_ref, pts_ref, feat_ref, out_ref):
    c = pts_ref.shape[1]
    rows, bn = out_ref.shape[1], out_ref.shape[2]
    out_ref[0] = jnp.broadcast_to(feat_ref[0], (rows, bn))
    out_ref[0, :c, :] = pts_ref[0]


def kernel(points, subject_garment_id, features):
    b, c, n = points.shape
    s, l = features.shape
    grid = (b, n // _BN)
    feats_pad = jnp.concatenate(
        [jnp.zeros((s, c), jnp.float32), features], axis=1
    ).reshape(s, c + l, 1)

    return pl.pallas_call(
        _body,
        grid_spec=pltpu.PrefetchScalarGridSpec(
            num_scalar_prefetch=1,
            grid=grid,
            in_specs=[
                pl.BlockSpec((1, c, _BN), lambda bi, ni, sid: (bi, 0, ni)),
                pl.BlockSpec((1, c + l, 1), lambda bi, ni, sid: (sid[bi], 0, 0)),
            ],
            out_specs=pl.BlockSpec((1, c + l, _BN), lambda bi, ni, sid: (bi, 0, ni)),
        ),
        out_shape=jax.ShapeDtypeStruct((b, c + l, n), jnp.float32),
    )(subject_garment_id, points, feats_pad)
